# baseline (speedup 1.0000x reference)
import functools
import math

import jax
import jax.numpy as jnp
from jax import lax
from jax.experimental import pallas as pl
from jax.experimental.pallas import tpu as pltpu
from jax.experimental.pallas import tpu_sc as plsc

_L = 16


@functools.cache
def _build(B0, S, V, D):
    try:
        info = plsc.get_sparse_core_info()
        NC, NS = info.num_cores, info.num_subcores
    except ValueError:
        NC, NS = 2, 16
    NW = NC * NS
    assert B0 == 128 * NW and D == 32 and S % 8 == 0
    J = D // 8
    SA = S // 8
    CH = 4
    NBUF = 3
    nchunks = S // CH
    scale = math.sqrt(D)
    mesh = plsc.VectorSubcoreMesh(
        core_axis_name="c", subcore_axis_name="s", num_cores=NC, num_subcores=NS
    )

    @functools.partial(
        pl.kernel,
        out_type=jax.ShapeDtypeStruct((S, J, NW, 8, 128), jnp.float32),
        mesh=mesh,
        scratch_types=[pltpu.VMEM((SA, 8, 128), jnp.int32)]
        + [pltpu.VMEM((CH * 128, D), jnp.float32) for _ in range(NBUF)]
        + [pltpu.VMEM((CH, J, 8, 129), jnp.float32) for _ in range(NBUF)]
        + [pltpu.SemaphoreType.DMA((NBUF,)), pltpu.SemaphoreType.DMA((NBUF,))],
        compiler_params=pltpu.CompilerParams(
            use_tc_tiling_on_sc=False, needs_layout_passes=False
        ),
    )
    def emb(tok_hbm, table_hbm, out_hbm, tokv, *bufs):
        rows_v = bufs[:NBUF]
        stg_v = bufs[NBUF : 2 * NBUF]
        gsem, osem = bufs[2 * NBUF], bufs[2 * NBUF + 1]
        wid = lax.axis_index("s") * NC + lax.axis_index("c")
        iota = lax.iota(jnp.int32, _L)

        pltpu.sync_copy(tok_hbm.at[:, wid], tokv)

        def idx_ref(c, sl):
            s = c * CH + sl
            return tokv.at[s // 8, s % 8]

        def gather_start(c, b):
            for sl in range(CH):
                pltpu.async_copy(
                    table_hbm.at[idx_ref(c, sl)],
                    rows_v[b].at[pl.ds(128 * sl, 128)],
                    gsem.at[b],
                )

        def gather_wait(c, b):
            for sl in range(CH):
                pltpu.make_async_copy(
                    table_hbm.at[idx_ref(c, sl)],
                    rows_v[b].at[pl.ds(128 * sl, 128)],
                    gsem.at[b],
                ).wait()

        def out_start(c, b):
            pltpu.async_copy(
                stg_v[b].at[:, :, :, pl.ds(0, 128)],
                out_hbm.at[pl.ds(CH * c, CH), :, wid],
                osem.at[b],
            )

        def out_wait(b):
            pltpu.make_async_copy(
                stg_v[b].at[:, :, :, pl.ds(0, 128)],
                out_hbm.at[pl.ds(0, CH), :, wid],
                osem.at[b],
            ).wait()

        j_vec = [(iota + h * _L) // 8 for h in range(D // _L)]
        dlo_vec = [(iota + h * _L) % 8 for h in range(D // _L)]

        def transpose_scale(b):
            @pl.loop(0, CH * 128, unroll=4)
            def per_i(i):
                sl = jnp.full((_L,), 0, jnp.int32) + (i // 128)
                bl = jnp.full((_L,), 0, jnp.int32) + (i % 128)
                for h in range(D // _L):
                    v = rows_v[b][i, pl.ds(h * _L, _L)]
                    plsc.store_scatter(
                        stg_v[b], [sl, j_vec[h], dlo_vec[h], bl], v * scale
                    )

        for b in range(NBUF):
            gather_start(b, b)

        for b in range(NBUF):
            gather_wait(b, b)
            transpose_scale(b)
            out_start(b, b)
            gather_start(b + NBUF, b)

        @pl.loop(1, nchunks // NBUF)
        def outer(g):
            for b in range(NBUF):
                c = g * NBUF + b
                gather_wait(c, b)
                out_wait(b)
                transpose_scale(b)
                out_start(c, b)

                @pl.when(c + NBUF < nchunks)
                def _():
                    gather_start(c + NBUF, b)

        for c in range(NBUF * (nchunks // NBUF), nchunks):
            b = c % NBUF
            gather_wait(c, b)
            out_wait(b)
            transpose_scale(b)
            out_start(c, b)

        for b in range(NBUF):
            out_wait(b)

    return emb


def kernel(batched_tokens, table):
    B0, S = batched_tokens.shape
    V, D = table.shape
    tok_native = (
        batched_tokens.astype(jnp.int32)
        .T.reshape(S // 8, 8, B0 // 128, 128)
        .transpose(0, 2, 1, 3)
    )
    r = _build(B0, S, V, D)(tok_native, table)
    return r.transpose(2, 4, 0, 1, 3).reshape(B0, S, D)


# speedup vs baseline: 1.0217x; 1.0217x over previous
import functools
import math

import jax
import jax.numpy as jnp
from jax import lax
from jax.experimental import pallas as pl
from jax.experimental.pallas import tpu as pltpu
from jax.experimental.pallas import tpu_sc as plsc

_L = 16


@functools.cache
def _build(B0, S, V, D):
    try:
        info = plsc.get_sparse_core_info()
        NC, NS = info.num_cores, info.num_subcores
    except ValueError:
        NC, NS = 2, 16
    NW = NC * NS
    assert B0 == 128 * NW and D == 32 and S % 8 == 0
    J = D // 8
    SA = S // 8
    NBUF = 4
    nsteps = S // NBUF
    scale = math.sqrt(D)
    mesh = plsc.VectorSubcoreMesh(
        core_axis_name="c", subcore_axis_name="s", num_cores=NC, num_subcores=NS
    )

    @functools.partial(
        pl.kernel,
        out_type=jax.ShapeDtypeStruct((S, J, NW, 8, 128), jnp.float32),
        mesh=mesh,
        scratch_types=[pltpu.VMEM((SA, 8, 128), jnp.int32)]
        + [pltpu.VMEM((128, D), jnp.float32) for _ in range(NBUF)]
        + [pltpu.VMEM((D, 129), jnp.float32) for _ in range(NBUF)]
        + [pltpu.SemaphoreType.DMA((NBUF,)), pltpu.SemaphoreType.DMA((NBUF,))],
        compiler_params=pltpu.CompilerParams(
            use_tc_tiling_on_sc=False, needs_layout_passes=False
        ),
    )
    def emb(tok_hbm, table_hbm, out_hbm, tokv, *bufs):
        rows_v = bufs[:NBUF]
        stg_v = bufs[NBUF : 2 * NBUF]
        gsem, osem = bufs[2 * NBUF], bufs[2 * NBUF + 1]
        wid = lax.axis_index("s") * NC + lax.axis_index("c")
        iota = lax.iota(jnp.int32, _L)

        pltpu.sync_copy(tok_hbm.at[:, wid], tokv)

        def idx_ref(s):
            return tokv.at[s // 8, s % 8]

        def gather_start(s, b):
            pltpu.async_copy(table_hbm.at[idx_ref(s)], rows_v[b], gsem.at[b])

        def gather_wait(s, b):
            pltpu.make_async_copy(
                table_hbm.at[idx_ref(s)], rows_v[b], gsem.at[b]
            ).wait()

        def out_start(s, b):
            for j in range(J):
                pltpu.async_copy(
                    stg_v[b].at[pl.ds(8 * j, 8), pl.ds(0, 128)],
                    out_hbm.at[s, j, wid],
                    osem.at[b],
                )

        def out_wait(b):
            for j in range(J):
                pltpu.make_async_copy(
                    stg_v[b].at[pl.ds(8 * j, 8), pl.ds(0, 128)],
                    out_hbm.at[0, j, wid],
                    osem.at[b],
                ).wait()

        d_half = [iota + h * _L for h in range(D // _L)]

        def transpose_scale(b):
            @pl.loop(0, 128, unroll=8)
            def per_i(i):
                col = jnp.full((_L,), 0, jnp.int32) + i
                for h in range(D // _L):
                    v = rows_v[b][i, pl.ds(h * _L, _L)]
                    plsc.store_scatter(stg_v[b], [d_half[h], col], v * scale)

        for b in range(NBUF):
            gather_start(b, b)

        for b in range(NBUF):
            gather_wait(b, b)
            transpose_scale(b)
            out_start(b, b)
            gather_start(b + NBUF, b)

        @pl.loop(1, nsteps)
        def outer(g):
            for b in range(NBUF):
                s = g * NBUF + b
                gather_wait(s, b)
                out_wait(b)
                transpose_scale(b)
                out_start(s, b)

                @pl.when(s + NBUF < S)
                def _():
                    gather_start(s + NBUF, b)

        for b in range(NBUF):
            out_wait(b)

    return emb


def kernel(batched_tokens, table):
    B0, S = batched_tokens.shape
    V, D = table.shape
    tok_native = (
        batched_tokens.astype(jnp.int32)
        .T.reshape(S // 8, 8, B0 // 128, 128)
        .transpose(0, 2, 1, 3)
    )
    r = _build(B0, S, V, D)(tok_native, table)
    return r.transpose(2, 4, 0, 1, 3).reshape(B0, S, D)
